# baseline (speedup 1.0000x reference)
import jax
import jax.numpy as jnp
from jax.experimental import pallas as pl
from jax.experimental.pallas import tpu as pltpu

DIM_X, DIM_Y = 4, 4
D = DIM_X * DIM_Y
N = D * D
EMBED = 64
H = 256
B = 64
ITERS = 2
BB = 64


def _main_kernel(src_ref, dst_ref, xg_ref, xp_ref, table_ref, *rest):
    wb_refs, out_ref, a_ref = rest[:-2], rest[-2], rest[-1]
    ws = [wb_refs[2 * i][...] for i in range(9)]
    bs = [wb_refs[2 * i + 1][...][None, :] for i in range(9)]
    table = table_ref[...]

    @pl.when(pl.program_id(0) == 0)
    def _build_adjacency():
        s = src_ref[...][None, :]
        d = dst_ref[...][None, :]
        e = s.shape[1]
        dm = (jax.lax.broadcasted_iota(jnp.int32, (N, e), 0) == d).astype(jnp.float32)
        sm = (jax.lax.broadcasted_iota(jnp.int32, (N, e), 0) == s).astype(jnp.float32)
        a_ref[...] = jax.lax.dot_general(
            dm, sm, (((1,), (1,)), ((), ())),
            preferred_element_type=jnp.float32)

    a = a_ref[...]

    xg = xg_ref[...]
    onehot = (xg[:, :, None] == jax.lax.broadcasted_iota(
        jnp.int32, (BB, N, D + 1), 2)).astype(jnp.float32).reshape(BB * N, D + 1)

    w0a, w0b = ws[0][:EMBED], ws[0][EMBED:]
    tw = jnp.dot(table, w0a, preferred_element_type=jnp.float32)
    pre = jnp.dot(onehot, tw, preferred_element_type=jnp.float32) + bs[0]

    xt = xp_ref[...].astype(jnp.bfloat16)

    def dense(v, i, relu):
        v = jnp.dot(v.astype(jnp.bfloat16), ws[i].astype(jnp.bfloat16),
                    preferred_element_type=jnp.float32) + bs[i]
        return jnp.maximum(v, 0.0) if relu else v

    w0b_bf = w0b.astype(jnp.bfloat16)
    w8t_bf = ws[8].astype(jnp.bfloat16)
    for it in range(ITERS):
        xw = jnp.concatenate(
            [jax.lax.dot_general(xt[b], w0b_bf, (((0,), (0,)), ((), ())),
                                 preferred_element_type=jnp.float32)
             for b in range(BB)], axis=0)
        h = jnp.maximum(pre + xw, 0.0)
        h = dense(h, 1, True)
        h = dense(h, 2, True)
        h3 = h.reshape(BB, N, H)
        agg = jnp.concatenate(
            [jnp.dot(a, h3[b], preferred_element_type=jnp.float32)
             for b in range(BB)], axis=0)
        h = dense(agg, 3, True)
        h = dense(h, 4, True)
        h = dense(h, 5, True)
        h = dense(h, 6, True)
        h = dense(h, 7, True)
        logits = jax.lax.dot_general(
            h.astype(jnp.bfloat16), w8t_bf, (((1,), (1,)), ((), ())),
            preferred_element_type=jnp.float32) + bs[8]
        lt = jnp.transpose(logits.reshape(BB, N, D), (0, 2, 1))
        out_ref[it] = lt
        m = jnp.max(lt, axis=1, keepdims=True)
        ex = jnp.exp(lt - m)
        xt = (ex / jnp.sum(ex, axis=1, keepdims=True)).astype(jnp.bfloat16)


def kernel(x_grid_form, x_prob_form, iters, embed_table, src, dst,
           embed_mlp, rel_mlp, decode_mlp):
    del iters
    e = src.shape[0]

    wbs = []
    for params in (embed_mlp, rel_mlp, decode_mlp):
        for w, bvec in params:
            wbs.append(w)
            wbs.append(bvec)
    wbs[16] = wbs[16].T

    full = lambda shape: pl.BlockSpec(shape, lambda *_: (0,) * len(shape))
    in_specs = [
        full((e,)),
        full((e,)),
        pl.BlockSpec((BB, N), lambda i: (i, 0)),
        pl.BlockSpec((BB, D, N), lambda i: (i, 0, 0)),
        full(embed_table.shape),
    ] + [full(w.shape) for w in wbs]

    outs = pl.pallas_call(
        _main_kernel,
        grid=(B // BB,),
        in_specs=in_specs,
        out_specs=pl.BlockSpec((ITERS, BB, D, N), lambda i: (0, i, 0, 0)),
        out_shape=jax.ShapeDtypeStruct((ITERS, B, D, N), jnp.float32),
        scratch_shapes=[pltpu.VMEM((N, N), jnp.float32)],
    )(src, dst, x_grid_form, x_prob_form.transpose(0, 2, 1), embed_table, *wbs)

    return outs.transpose(0, 1, 3, 2)


# speedup vs baseline: 1.0692x; 1.0692x over previous
import jax
import jax.numpy as jnp
from jax.experimental import pallas as pl
from jax.experimental.pallas import tpu as pltpu

DIM_X, DIM_Y = 4, 4
D = DIM_X * DIM_Y
N = D * D
EMBED = 64
H = 256
B = 64
ITERS = 2
BB = 32


def _main_kernel(src_ref, dst_ref, xg_ref, xp_ref, table_ref, *rest):
    wb_refs, out_ref, a_ref = rest[:-2], rest[-2], rest[-1]
    ws = [wb_refs[2 * i][...] for i in range(9)]
    bs = [wb_refs[2 * i + 1][...][None, :] for i in range(9)]
    table = table_ref[...]

    @pl.when(pl.program_id(0) == 0)
    def _build_adjacency():
        s = src_ref[...][None, :]
        d = dst_ref[...][None, :]
        e = s.shape[1]
        dm = (jax.lax.broadcasted_iota(jnp.int32, (N, e), 0) == d).astype(jnp.bfloat16)
        sm = (jax.lax.broadcasted_iota(jnp.int32, (N, e), 0) == s).astype(jnp.bfloat16)
        a_ref[...] = jax.lax.dot_general(
            dm, sm, (((1,), (1,)), ((), ())),
            preferred_element_type=jnp.float32)

    a_bf = a_ref[...].astype(jnp.bfloat16)

    xg = xg_ref[...]
    onehot = (xg[:, :, None] == jax.lax.broadcasted_iota(
        jnp.int32, (BB, N, D + 1), 2)).astype(jnp.float32).reshape(BB * N, D + 1)

    w0a, w0b = ws[0][:EMBED], ws[0][EMBED:]
    tw = jnp.dot(table, w0a, preferred_element_type=jnp.float32)
    pre = jnp.dot(onehot, tw, preferred_element_type=jnp.float32) + bs[0]

    xt = xp_ref[...].astype(jnp.bfloat16)

    def dense(v, i, relu):
        v = jnp.dot(v.astype(jnp.bfloat16), ws[i].astype(jnp.bfloat16),
                    preferred_element_type=jnp.float32) + bs[i]
        return jnp.maximum(v, 0.0) if relu else v

    w0b_bf = w0b.astype(jnp.bfloat16)
    w8t_bf = ws[8].astype(jnp.bfloat16)
    for it in range(ITERS):
        xw = jnp.concatenate(
            [jax.lax.dot_general(xt[b], w0b_bf, (((0,), (0,)), ((), ())),
                                 preferred_element_type=jnp.float32)
             for b in range(BB)], axis=0)
        h = jnp.maximum(pre + xw, 0.0)
        h = dense(h, 1, True)
        h = dense(h, 2, True)
        h3 = h.astype(jnp.bfloat16).reshape(BB, N, H)
        agg = jnp.concatenate(
            [jnp.dot(a_bf, h3[b], preferred_element_type=jnp.float32)
             for b in range(BB)], axis=0)
        h = dense(agg, 3, True)
        h = dense(h, 4, True)
        h = dense(h, 5, True)
        h = dense(h, 6, True)
        h = dense(h, 7, True)
        logits = jax.lax.dot_general(
            h.astype(jnp.bfloat16), w8t_bf, (((1,), (1,)), ((), ())),
            preferred_element_type=jnp.float32) + bs[8]
        lt = jnp.transpose(logits.reshape(BB, N, D), (0, 2, 1))
        out_ref[it] = lt
        m = jnp.max(lt, axis=1, keepdims=True)
        ex = jnp.exp(lt - m)
        xt = (ex / jnp.sum(ex, axis=1, keepdims=True)).astype(jnp.bfloat16)


def kernel(x_grid_form, x_prob_form, iters, embed_table, src, dst,
           embed_mlp, rel_mlp, decode_mlp):
    del iters
    e = src.shape[0]

    wbs = []
    for params in (embed_mlp, rel_mlp, decode_mlp):
        for w, bvec in params:
            wbs.append(w)
            wbs.append(bvec)
    wbs[16] = wbs[16].T

    full = lambda shape: pl.BlockSpec(shape, lambda *_: (0,) * len(shape))
    in_specs = [
        full((e,)),
        full((e,)),
        pl.BlockSpec((BB, N), lambda i: (i, 0)),
        pl.BlockSpec((BB, D, N), lambda i: (i, 0, 0)),
        full(embed_table.shape),
    ] + [full(w.shape) for w in wbs]

    outs = pl.pallas_call(
        _main_kernel,
        grid=(B // BB,),
        in_specs=in_specs,
        out_specs=pl.BlockSpec((ITERS, BB, D, N), lambda i: (0, i, 0, 0)),
        out_shape=jax.ShapeDtypeStruct((ITERS, B, D, N), jnp.float32),
        scratch_shapes=[pltpu.VMEM((N, N), jnp.float32)],
    )(src, dst, x_grid_form, x_prob_form.transpose(0, 2, 1), embed_table, *wbs)

    return outs.transpose(0, 1, 3, 2)
